# baseline (speedup 1.0000x reference)
import functools

import jax
import jax.numpy as jnp
import numpy as np
from jax import lax
from jax.experimental import pallas as pl
from jax.experimental.pallas import tpu as pltpu
from jax.experimental.pallas import tpu_sc as plsc

B = 128
A = 8192
HL_BEAM = 8
LL_BEAM = 8
BEAM = 16
HRL_A = 0.6
NEG_INF = float("-inf")
BB = 16

_SEL = np.zeros((HL_BEAM, BB, BB * HL_BEAM), np.float32)
for _h in range(HL_BEAM):
    for _b in range(BB):
        _SEL[_h, _b, _b * HL_BEAM + _h] = 1.0


def _topk8_inline(x):
    r, a = x.shape
    col = lax.broadcasted_iota(jnp.int32, (r, a), 1)
    vals, ids = [], []
    for _ in range(HL_BEAM):
        m = jnp.max(x, axis=1, keepdims=True)
        hit = x == m
        idx = jnp.min(jnp.where(hit, col, a), axis=1, keepdims=True)
        vals.append(m)
        ids.append(idx)
        x = jnp.where(col == idx, NEG_INF, x)
    return jnp.concatenate(vals, axis=1), jnp.concatenate(ids, axis=1)


def _fused_body(hl_ref, ll_ref, sel_ref, beam_ref, hlg_ref, llg_ref, rel_ref,
                llrow_ref):
    step = pl.program_id(0)
    hlv, hlid = _topk8_inline(hl_ref[...])
    llv, llid = _topk8_inline(ll_ref[...])
    llid_f = llid.astype(jnp.float32)
    vparts, iparts = [], []
    for hh in range(HL_BEAM):
        sel_m = sel_ref[hh * BB:(hh + 1) * BB, :]
        vparts.append(jax.lax.dot(sel_m, llv,
                                  preferred_element_type=jnp.float32))
        iparts.append(jax.lax.dot(sel_m, llid_f,
                                  preferred_element_type=jnp.float32))
    llv64 = jnp.concatenate(vparts, axis=1)
    llid64 = jnp.concatenate(iparts, axis=1).astype(jnp.int32)

    w = HL_BEAM * LL_BEAM
    j = lax.broadcasted_iota(jnp.int32, (BB, w), 1)
    b = lax.broadcasted_iota(jnp.int32, (BB, w), 0) + step * BB
    h = j // LL_BEAM

    hl_t = jnp.zeros((BB, w), jnp.float32)
    hlid_t = jnp.zeros((BB, w), jnp.int32)
    for hh in range(HL_BEAM):
        sel = h == hh
        hl_t = jnp.where(sel, hlv[:, hh:hh + 1], hl_t)
        hlid_t = jnp.where(sel, hlid[:, hh:hh + 1], hlid_t)

    cmb = (1.0 - HRL_A) * llv64 + HRL_A * hl_t
    rel_row = b * (2 * A) + (hlid_t >> 7) * 256 + (hlid_t & 127)
    ll_row = (b * HL_BEAM + h) * (2 * A) + (llid64 >> 7) * 256 + (llid64 & 127)

    beams, hlgs, llgs, relrows, llrows = [], [], [], [], []
    for _ in range(BEAM):
        m = jnp.max(cmb, axis=1, keepdims=True)
        hit = cmb == m
        idx = jnp.min(jnp.where(hit, j, w), axis=1, keepdims=True)
        sel = j == idx
        beams.append(m)
        hlgs.append(jnp.sum(jnp.where(sel, hl_t, 0.0), axis=1, keepdims=True))
        llgs.append(jnp.sum(jnp.where(sel, llv64, 0.0), axis=1, keepdims=True))
        relrows.append(jnp.sum(jnp.where(sel, rel_row, 0), axis=1, keepdims=True))
        llrows.append(jnp.sum(jnp.where(sel, ll_row, 0), axis=1, keepdims=True))
        cmb = jnp.where(sel, NEG_INF, cmb)

    beam_ref[...] = jnp.concatenate(beams, axis=1)
    hlg_ref[...] = jnp.concatenate(hlgs, axis=1)
    llg_ref[...] = jnp.concatenate(llgs, axis=1)
    rel_ref[...] = jnp.concatenate(relrows, axis=1)
    llrow_ref[...] = jnp.concatenate(llrows, axis=1)


def _fused(logits_hl, logits_ll):
    grid = B // BB
    return pl.pallas_call(
        _fused_body,
        grid=(grid,),
        in_specs=[
            pl.BlockSpec((BB, A), lambda i: (i, 0)),
            pl.BlockSpec((BB * HL_BEAM, A), lambda i: (i, 0)),
            pl.BlockSpec((HL_BEAM * BB, BB * HL_BEAM), lambda i: (0, 0)),
        ],
        out_specs=[
            pl.BlockSpec((BB, BEAM), lambda i: (i, 0)),
            pl.BlockSpec((BB, BEAM), lambda i: (i, 0)),
            pl.BlockSpec((BB, BEAM), lambda i: (i, 0)),
            pl.BlockSpec((BB, BEAM), lambda i: (i, 0)),
            pl.BlockSpec((BB, BEAM), lambda i: (i, 0)),
        ],
        out_shape=[
            jax.ShapeDtypeStruct((B, BEAM), jnp.float32),
            jax.ShapeDtypeStruct((B, BEAM), jnp.float32),
            jax.ShapeDtypeStruct((B, BEAM), jnp.float32),
            jax.ShapeDtypeStruct((B, BEAM), jnp.int32),
            jax.ShapeDtypeStruct((B, BEAM), jnp.int32),
        ],
    )(logits_hl, logits_ll, jnp.asarray(_SEL.reshape(HL_BEAM * BB, BB * HL_BEAM)))


def _take1(v, i_scalar):
    idx = jnp.broadcast_to(i_scalar, (16,))[:, None]
    return lax.gather(
        v, idx,
        lax.GatherDimensionNumbers(
            offset_dims=(), collapsed_slice_dims=(0,), start_index_map=(0,)),
        (1,),
        mode=lax.GatherScatterMode.PROMISE_IN_BOUNDS)


def _sc_gather(rel_eidx, ll_eidx, hl_tab, ll_tab):
    info = plsc.get_sparse_core_info()
    nc, ns, nl = info.num_cores, info.num_subcores, info.num_lanes
    nw = nc * ns
    n = rel_eidx.shape[0]
    per = n // nw
    mesh = plsc.VectorSubcoreMesh(core_axis_name="c", subcore_axis_name="s")

    @functools.partial(
        pl.kernel,
        mesh=mesh,
        out_type=[
            jax.ShapeDtypeStruct((n,), jnp.int32),
            jax.ShapeDtypeStruct((n,), jnp.int32),
            jax.ShapeDtypeStruct((n,), jnp.int32),
        ],
        scratch_types=[
            pltpu.VMEM((per,), jnp.int32),
            pltpu.VMEM((per,), jnp.int32),
            pltpu.VMEM((per,), jnp.int32),
            pltpu.VMEM((per,), jnp.int32),
            pltpu.VMEM((per,), jnp.int32),
            pltpu.VMEM((per, 128), jnp.int32),
            pltpu.VMEM((per, 128), jnp.int32),
            pltpu.VMEM((per, 128), jnp.int32),
            pltpu.VMEM((per,), jnp.int32),
            pltpu.VMEM((per,), jnp.int32),
            pltpu.VMEM((per,), jnp.int32),
            pltpu.SemaphoreType.DMA,
            pltpu.SemaphoreType.DMA,
            pltpu.SemaphoreType.DMA,
        ],
    )
    def k(rel_idx_hbm, ll_idx_hbm, hl_tab_hbm, ll_tab_hbm,
          out_rel, out_ent, out_time,
          eidx1, eidx2, row1, row2, row3, rows1, rows2, rows3,
          o1, o2, o3, sem1, sem2, sem3):
        wid = lax.axis_index("s") * nc + lax.axis_index("c")
        base = wid * per
        pltpu.sync_copy(rel_idx_hbm.at[pl.ds(base, per)], eidx1)
        pltpu.sync_copy(ll_idx_hbm.at[pl.ds(base, per)], eidx2)
        for c in range(per // nl):
            s = pl.ds(c * nl, nl)
            row1[s] = eidx1[s] >> 7
            r2 = eidx2[s] >> 7
            row2[s] = r2
            row3[s] = r2 + 1
        c1 = pltpu.async_copy(hl_tab_hbm.at[row1], rows1, sem1)
        c2 = pltpu.async_copy(ll_tab_hbm.at[row2], rows2, sem2)
        c3 = pltpu.async_copy(ll_tab_hbm.at[row3], rows3, sem3)
        c1.wait()
        c2.wait()
        c3.wait()
        iota16 = lax.broadcasted_iota(jnp.int32, (nl,), 0)
        for g in range(per // nl):
            s = pl.ds(g * nl, nl)
            e1 = eidx1[s]
            e2 = eidx2[s]
            sub1 = (e1 & 127) >> 4
            off1 = e1 & 15
            sub2 = (e2 & 127) >> 4
            off2 = e2 & 15
            a1 = jnp.zeros((nl,), jnp.int32)
            a2 = jnp.zeros((nl,), jnp.int32)
            a3 = jnp.zeros((nl,), jnp.int32)
            for kk in range(nl):
                i = g * nl + kk
                hit = iota16 == kk
                v1 = rows1[i, pl.ds(sub1[kk] * nl, nl)]
                v2 = rows2[i, pl.ds(sub2[kk] * nl, nl)]
                v3 = rows3[i, pl.ds(sub2[kk] * nl, nl)]
                a1 = jnp.where(hit, _take1(v1, off1[kk]), a1)
                a2 = jnp.where(hit, _take1(v2, off2[kk]), a2)
                a3 = jnp.where(hit, _take1(v3, off2[kk]), a3)
            o1[s] = a1
            o2[s] = a2
            o3[s] = a3
        pltpu.sync_copy(o1, out_rel.at[pl.ds(base, per)])
        pltpu.sync_copy(o2, out_ent.at[pl.ds(base, per)])
        pltpu.sync_copy(o3, out_time.at[pl.ds(base, per)])

    return k(rel_eidx, ll_eidx, hl_tab, ll_tab)


def kernel(logits_hl, hl_space, logits_ll, ll_space):
    beam, hl_g, ll_g, rel_row, ll_row = _fused(logits_hl, logits_ll)

    hl_tab = (hl_space.reshape(B, A // 128, 128, 2)
              .transpose(0, 1, 3, 2).reshape(B * A * 2 // 128, 128))
    ll_tab = (ll_space.reshape(B * HL_BEAM, A // 128, 128, 2)
              .transpose(0, 1, 3, 2).reshape(B * HL_BEAM * A * 2 // 128, 128))
    rels, ents, times = _sc_gather(
        rel_row.reshape(-1), ll_row.reshape(-1), hl_tab, ll_tab)

    return (
        beam,
        hl_g.reshape(-1),
        ll_g.reshape(-1),
        ents,
        times,
        rels,
    )


# speedup vs baseline: 1.4314x; 1.4314x over previous
import functools

import jax
import jax.numpy as jnp
from jax import lax
from jax.experimental import pallas as pl
from jax.experimental.pallas import tpu as pltpu
from jax.experimental.pallas import tpu_sc as plsc

B = 128
A = 8192
HL_BEAM = 8
LL_BEAM = 8
BEAM = 16
HRL_A = 0.6
NEG_INF = float("-inf")
INT_MIN = -2147483648
LANES = 128
NSLAB = A // LANES


def _to_key(x):
    b = lax.bitcast_convert_type(x, jnp.int32)
    return b ^ ((b >> 31) & 0x7FFFFFFF)


def _from_key(k):
    b = k ^ ((k >> 31) & 0x7FFFFFFF)
    return lax.bitcast_convert_type(b, jnp.float32)


def _topk8_exact(x, r):
    col = lax.broadcasted_iota(jnp.int32, (r, A), 1)
    vals, ids = [], []
    for _ in range(HL_BEAM):
        m = jnp.max(x, axis=1, keepdims=True)
        hit = x == m
        idx = jnp.min(jnp.where(hit, col, A), axis=1, keepdims=True)
        vals.append(m)
        ids.append(idx)
        x = jnp.where(col == idx, NEG_INF, x)
    return jnp.concatenate(vals, axis=1), jnp.concatenate(ids, axis=1)


def _topk8_body(x_ref, vals_ref, ids_ref):
    r = x_ref.shape[0]
    sent = jnp.full((r, LANES), INT_MIN, jnp.int32)
    zero = jnp.zeros((r, LANES), jnp.int32)
    s1 = _to_key(x_ref[:, 0:LANES])
    ms1 = zero
    s2, s3, s4 = sent, sent, sent
    ms2, ms3, ms4 = zero, zero, zero
    for s in range(1, NSLAB):
        k = _to_key(x_ref[:, s * LANES:(s + 1) * LANES])
        g1 = k > s1
        g2 = k > s2
        g3 = k > s3
        g4 = k > s4
        s4 = jnp.where(g3, s3, jnp.where(g4, k, s4))
        ms4 = jnp.where(g3, ms3, jnp.where(g4, s, ms4))
        s3 = jnp.where(g2, s2, jnp.where(g3, k, s3))
        ms3 = jnp.where(g2, ms2, jnp.where(g3, s, ms3))
        s2 = jnp.where(g1, s1, jnp.where(g2, k, s2))
        ms2 = jnp.where(g1, ms1, jnp.where(g2, s, ms2))
        s1 = jnp.where(g1, k, s1)
        ms1 = jnp.where(g1, s, ms1)

    lane = lax.broadcasted_iota(jnp.int32, (r, LANES), 1)
    flag = jnp.zeros((), jnp.bool_)
    kvals, ids = [], []
    for it in range(HL_BEAM):
        m = jnp.max(s1, axis=1, keepdims=True)
        hit = s1 == m
        acand = ms1 * LANES + lane
        aidx = jnp.min(jnp.where(hit, acand, A), axis=1, keepdims=True)
        kvals.append(m)
        ids.append(aidx)
        flag = jnp.logical_or(flag, jnp.any(m == INT_MIN))
        oneh = lane == (aidx & (LANES - 1))
        s1 = jnp.where(oneh, s2, s1)
        ms1 = jnp.where(oneh, ms2, ms1)
        s2 = jnp.where(oneh, s3, s2)
        ms2 = jnp.where(oneh, ms3, ms2)
        s3 = jnp.where(oneh, s4, s3)
        ms3 = jnp.where(oneh, ms4, ms3)
        s4 = jnp.where(oneh, INT_MIN, s4)
        if it < HL_BEAM - 1:
            flag = jnp.logical_or(flag, jnp.any(s1 == INT_MIN))

    vals_ref[...] = _from_key(jnp.concatenate(kvals, axis=1))
    ids_ref[...] = jnp.concatenate(ids, axis=1)

    @pl.when(flag)
    def _slow():
        v, i = _topk8_exact(x_ref[...], r)
        vals_ref[...] = v
        ids_ref[...] = i


def _topk8(x, row_block):
    rows = x.shape[0]
    grid = rows // row_block
    return pl.pallas_call(
        _topk8_body,
        grid=(grid,),
        in_specs=[pl.BlockSpec((row_block, A), lambda i: (i, 0))],
        out_specs=[
            pl.BlockSpec((row_block, HL_BEAM), lambda i: (i, 0)),
            pl.BlockSpec((row_block, HL_BEAM), lambda i: (i, 0)),
        ],
        out_shape=[
            jax.ShapeDtypeStruct((rows, HL_BEAM), jnp.float32),
            jax.ShapeDtypeStruct((rows, HL_BEAM), jnp.int32),
        ],
    )(x)


def _combine_body(hlv_ref, hlid_ref, llv_ref, llid_ref,
                  beam_ref, hlg_ref, llg_ref, relrow_ref, llrow_ref):
    hlv = hlv_ref[...]
    hlid = hlid_ref[...]
    llv = llv_ref[...]
    llid = llid_ref[...]
    b_, w = llv.shape
    j = lax.broadcasted_iota(jnp.int32, (b_, w), 1)
    b = lax.broadcasted_iota(jnp.int32, (b_, w), 0)
    h = j // LL_BEAM

    hl_t = jnp.zeros((b_, w), jnp.float32)
    hlid_t = jnp.zeros((b_, w), jnp.int32)
    for hh in range(HL_BEAM):
        sel = h == hh
        hl_t = jnp.where(sel, hlv[:, hh:hh + 1], hl_t)
        hlid_t = jnp.where(sel, hlid[:, hh:hh + 1], hlid_t)

    cmb = (1.0 - HRL_A) * llv + HRL_A * hl_t
    rel_row = b * (2 * A) + (hlid_t >> 7) * 256 + (hlid_t & 127)
    ll_row = (b * HL_BEAM + h) * (2 * A) + (llid >> 7) * 256 + (llid & 127)

    beams, hlgs, llgs, relrows, llrows = [], [], [], [], []
    for _ in range(BEAM):
        m = jnp.max(cmb, axis=1, keepdims=True)
        hit = cmb == m
        idx = jnp.min(jnp.where(hit, j, w), axis=1, keepdims=True)
        sel = j == idx
        beams.append(m)
        hlgs.append(jnp.sum(jnp.where(sel, hl_t, 0.0), axis=1, keepdims=True))
        llgs.append(jnp.sum(jnp.where(sel, llv, 0.0), axis=1, keepdims=True))
        relrows.append(jnp.sum(jnp.where(sel, rel_row, 0), axis=1, keepdims=True))
        llrows.append(jnp.sum(jnp.where(sel, ll_row, 0), axis=1, keepdims=True))
        cmb = jnp.where(sel, NEG_INF, cmb)

    beam_ref[...] = jnp.concatenate(beams, axis=1)
    hlg_ref[...] = jnp.concatenate(hlgs, axis=1)
    llg_ref[...] = jnp.concatenate(llgs, axis=1)
    relrow_ref[...] = jnp.concatenate(relrows, axis=1)
    llrow_ref[...] = jnp.concatenate(llrows, axis=1)


def _combine(hlv, hlid, llv64, llid64):
    return pl.pallas_call(
        _combine_body,
        out_shape=[
            jax.ShapeDtypeStruct((B, BEAM), jnp.float32),
            jax.ShapeDtypeStruct((B, BEAM), jnp.float32),
            jax.ShapeDtypeStruct((B, BEAM), jnp.float32),
            jax.ShapeDtypeStruct((B, BEAM), jnp.int32),
            jax.ShapeDtypeStruct((B, BEAM), jnp.int32),
        ],
    )(hlv, hlid, llv64, llid64)


def _take1(v, i_scalar):
    idx = jnp.broadcast_to(i_scalar, (16,))[:, None]
    return lax.gather(
        v, idx,
        lax.GatherDimensionNumbers(
            offset_dims=(), collapsed_slice_dims=(0,), start_index_map=(0,)),
        (1,),
        mode=lax.GatherScatterMode.PROMISE_IN_BOUNDS)


def _sc_gather(rel_eidx, ll_eidx, hl_tab, ll_tab):
    info = plsc.get_sparse_core_info()
    nc, ns, nl = info.num_cores, info.num_subcores, info.num_lanes
    nw = nc * ns
    n = rel_eidx.shape[0]
    per = n // nw
    mesh = plsc.VectorSubcoreMesh(core_axis_name="c", subcore_axis_name="s")

    @functools.partial(
        pl.kernel,
        mesh=mesh,
        out_type=[
            jax.ShapeDtypeStruct((n,), jnp.int32),
            jax.ShapeDtypeStruct((n,), jnp.int32),
            jax.ShapeDtypeStruct((n,), jnp.int32),
        ],
        scratch_types=[
            pltpu.VMEM((per,), jnp.int32),
            pltpu.VMEM((per,), jnp.int32),
            pltpu.VMEM((per,), jnp.int32),
            pltpu.VMEM((per,), jnp.int32),
            pltpu.VMEM((per,), jnp.int32),
            pltpu.VMEM((per, 128), jnp.int32),
            pltpu.VMEM((per, 128), jnp.int32),
            pltpu.VMEM((per, 128), jnp.int32),
            pltpu.VMEM((per,), jnp.int32),
            pltpu.VMEM((per,), jnp.int32),
            pltpu.VMEM((per,), jnp.int32),
            pltpu.SemaphoreType.DMA,
            pltpu.SemaphoreType.DMA,
            pltpu.SemaphoreType.DMA,
        ],
    )
    def k(rel_idx_hbm, ll_idx_hbm, hl_tab_hbm, ll_tab_hbm,
          out_rel, out_ent, out_time,
          eidx1, eidx2, row1, row2, row3, rows1, rows2, rows3,
          o1, o2, o3, sem1, sem2, sem3):
        wid = lax.axis_index("s") * nc + lax.axis_index("c")
        base = wid * per
        pltpu.sync_copy(rel_idx_hbm.at[pl.ds(base, per)], eidx1)
        pltpu.sync_copy(ll_idx_hbm.at[pl.ds(base, per)], eidx2)
        for c in range(per // nl):
            s = pl.ds(c * nl, nl)
            row1[s] = eidx1[s] >> 7
            r2 = eidx2[s] >> 7
            row2[s] = r2
            row3[s] = r2 + 1
        c1 = pltpu.async_copy(hl_tab_hbm.at[row1], rows1, sem1)
        c2 = pltpu.async_copy(ll_tab_hbm.at[row2], rows2, sem2)
        c3 = pltpu.async_copy(ll_tab_hbm.at[row3], rows3, sem3)
        c1.wait()
        c2.wait()
        c3.wait()
        iota16 = lax.broadcasted_iota(jnp.int32, (nl,), 0)
        for g in range(per // nl):
            s = pl.ds(g * nl, nl)
            e1 = eidx1[s]
            e2 = eidx2[s]
            sub1 = (e1 & 127) >> 4
            off1 = e1 & 15
            sub2 = (e2 & 127) >> 4
            off2 = e2 & 15
            a1 = jnp.zeros((nl,), jnp.int32)
            a2 = jnp.zeros((nl,), jnp.int32)
            a3 = jnp.zeros((nl,), jnp.int32)
            for kk in range(nl):
                i = g * nl + kk
                hit = iota16 == kk
                v1 = rows1[i, pl.ds(sub1[kk] * nl, nl)]
                v2 = rows2[i, pl.ds(sub2[kk] * nl, nl)]
                v3 = rows3[i, pl.ds(sub2[kk] * nl, nl)]
                a1 = jnp.where(hit, _take1(v1, off1[kk]), a1)
                a2 = jnp.where(hit, _take1(v2, off2[kk]), a2)
                a3 = jnp.where(hit, _take1(v3, off2[kk]), a3)
            o1[s] = a1
            o2[s] = a2
            o3[s] = a3
        pltpu.sync_copy(o1, out_rel.at[pl.ds(base, per)])
        pltpu.sync_copy(o2, out_ent.at[pl.ds(base, per)])
        pltpu.sync_copy(o3, out_time.at[pl.ds(base, per)])

    return k(rel_eidx, ll_eidx, hl_tab, ll_tab)


def kernel(logits_hl, hl_space, logits_ll, ll_space):
    hlv, hlid = _topk8(logits_hl, row_block=128)
    llv, llid = _topk8(logits_ll, row_block=128)

    llv64 = llv.reshape(B, HL_BEAM * LL_BEAM)
    llid64 = llid.reshape(B, HL_BEAM * LL_BEAM)

    beam, hl_g, ll_g, rel_row, ll_row = _combine(hlv, hlid, llv64, llid64)

    hl_tab = (hl_space.reshape(B, A // 128, 128, 2)
              .transpose(0, 1, 3, 2).reshape(B * A * 2 // 128, 128))
    ll_tab = (ll_space.reshape(B * HL_BEAM, A // 128, 128, 2)
              .transpose(0, 1, 3, 2).reshape(B * HL_BEAM * A * 2 // 128, 128))
    rels, ents, times = _sc_gather(
        rel_row.reshape(-1), ll_row.reshape(-1), hl_tab, ll_tab)

    return (
        beam,
        hl_g.reshape(-1),
        ll_g.reshape(-1),
        ents,
        times,
        rels,
    )


# speedup vs baseline: 1.4732x; 1.0292x over previous
import functools

import jax
import jax.numpy as jnp
from jax import lax
from jax.experimental import pallas as pl
from jax.experimental.pallas import tpu as pltpu
from jax.experimental.pallas import tpu_sc as plsc

B = 128
A = 8192
HL_BEAM = 8
LL_BEAM = 8
BEAM = 16
HRL_A = 0.6
NEG_INF = float("-inf")
INT_MIN = -2147483648
LANES = 128
NSLAB = A // LANES


def _to_key(x):
    b = lax.bitcast_convert_type(x, jnp.int32)
    return b ^ ((b >> 31) & 0x7FFFFFFF)


def _from_key(k):
    b = k ^ ((k >> 31) & 0x7FFFFFFF)
    return lax.bitcast_convert_type(b, jnp.float32)


def _topk8_exact(x, r):
    col = lax.broadcasted_iota(jnp.int32, (r, A), 1)
    vals, ids = [], []
    for _ in range(HL_BEAM):
        m = jnp.max(x, axis=1, keepdims=True)
        hit = x == m
        idx = jnp.min(jnp.where(hit, col, A), axis=1, keepdims=True)
        vals.append(m)
        ids.append(idx)
        x = jnp.where(col == idx, NEG_INF, x)
    return jnp.concatenate(vals, axis=1), jnp.concatenate(ids, axis=1)


def _topk8_body(x_ref, vals_ref, ids_ref):
    r = x_ref.shape[0]
    sent = jnp.full((r, LANES), INT_MIN, jnp.int32)
    zero = jnp.zeros((r, LANES), jnp.int32)
    s1 = _to_key(x_ref[:, 0:LANES])
    ms1 = zero
    s2, s3, s4 = sent, sent, sent
    ms2, ms3, ms4 = zero, zero, zero
    for s in range(1, NSLAB):
        k = _to_key(x_ref[:, s * LANES:(s + 1) * LANES])
        g1 = k > s1
        g2 = k > s2
        g3 = k > s3
        g4 = k > s4
        s4 = jnp.where(g3, s3, jnp.where(g4, k, s4))
        ms4 = jnp.where(g3, ms3, jnp.where(g4, s, ms4))
        s3 = jnp.where(g2, s2, jnp.where(g3, k, s3))
        ms3 = jnp.where(g2, ms2, jnp.where(g3, s, ms3))
        s2 = jnp.where(g1, s1, jnp.where(g2, k, s2))
        ms2 = jnp.where(g1, ms1, jnp.where(g2, s, ms2))
        s1 = jnp.where(g1, k, s1)
        ms1 = jnp.where(g1, s, ms1)

    lane = lax.broadcasted_iota(jnp.int32, (r, LANES), 1)
    flag = jnp.zeros((), jnp.bool_)
    kvals, ids = [], []
    for it in range(HL_BEAM):
        m = jnp.max(s1, axis=1, keepdims=True)
        hit = s1 == m
        acand = ms1 * LANES + lane
        aidx = jnp.min(jnp.where(hit, acand, A), axis=1, keepdims=True)
        kvals.append(m)
        ids.append(aidx)
        flag = jnp.logical_or(flag, jnp.any(m == INT_MIN))
        oneh = lane == (aidx & (LANES - 1))
        s1 = jnp.where(oneh, s2, s1)
        ms1 = jnp.where(oneh, ms2, ms1)
        s2 = jnp.where(oneh, s3, s2)
        ms2 = jnp.where(oneh, ms3, ms2)
        s3 = jnp.where(oneh, s4, s3)
        ms3 = jnp.where(oneh, ms4, ms3)
        s4 = jnp.where(oneh, INT_MIN, s4)
        if it < HL_BEAM - 1:
            flag = jnp.logical_or(flag, jnp.any(s1 == INT_MIN))

    vals_ref[...] = _from_key(jnp.concatenate(kvals, axis=1))
    ids_ref[...] = jnp.concatenate(ids, axis=1)

    @pl.when(flag)
    def _slow():
        v, i = _topk8_exact(x_ref[...], r)
        vals_ref[...] = v
        ids_ref[...] = i


def _topk8(x, row_block):
    rows = x.shape[0]
    grid = rows // row_block
    return pl.pallas_call(
        _topk8_body,
        grid=(grid,),
        in_specs=[pl.BlockSpec((row_block, A), lambda i: (i, 0))],
        out_specs=[
            pl.BlockSpec((row_block, HL_BEAM), lambda i: (i, 0)),
            pl.BlockSpec((row_block, HL_BEAM), lambda i: (i, 0)),
        ],
        out_shape=[
            jax.ShapeDtypeStruct((rows, HL_BEAM), jnp.float32),
            jax.ShapeDtypeStruct((rows, HL_BEAM), jnp.int32),
        ],
    )(x)


def _combine_body(hlv_ref, hlid_ref, llv_ref, llid_ref,
                  beam_ref, hlg_ref, llg_ref, relrow_ref, llrow_ref):
    hlv = hlv_ref[...]
    hlid = hlid_ref[...]
    llv = llv_ref[...]
    llid = llid_ref[...]
    b_, w = llv.shape
    j = lax.broadcasted_iota(jnp.int32, (b_, w), 1)
    b = lax.broadcasted_iota(jnp.int32, (b_, w), 0)
    h = j // LL_BEAM

    hl_t = jnp.zeros((b_, w), jnp.float32)
    hlid_t = jnp.zeros((b_, w), jnp.int32)
    for hh in range(HL_BEAM):
        sel = h == hh
        hl_t = jnp.where(sel, hlv[:, hh:hh + 1], hl_t)
        hlid_t = jnp.where(sel, hlid[:, hh:hh + 1], hlid_t)

    cmb = (1.0 - HRL_A) * llv + HRL_A * hl_t
    rel_row = b * (2 * A) + (hlid_t >> 7) * 256 + (hlid_t & 127)
    ll_row = (b * HL_BEAM + h) * (2 * A) + (llid >> 7) * 256 + (llid & 127)

    beams, hlgs, llgs, relrows, llrows = [], [], [], [], []
    for _ in range(BEAM):
        m = jnp.max(cmb, axis=1, keepdims=True)
        hit = cmb == m
        idx = jnp.min(jnp.where(hit, j, w), axis=1, keepdims=True)
        sel = j == idx
        beams.append(m)
        hlgs.append(jnp.sum(jnp.where(sel, hl_t, 0.0), axis=1, keepdims=True))
        llgs.append(jnp.sum(jnp.where(sel, llv, 0.0), axis=1, keepdims=True))
        relrows.append(jnp.sum(jnp.where(sel, rel_row, 0), axis=1, keepdims=True))
        llrows.append(jnp.sum(jnp.where(sel, ll_row, 0), axis=1, keepdims=True))
        cmb = jnp.where(sel, NEG_INF, cmb)

    beam_ref[...] = jnp.concatenate(beams, axis=1)
    hlg_ref[...] = jnp.concatenate(hlgs, axis=1)
    llg_ref[...] = jnp.concatenate(llgs, axis=1)
    relrow_ref[...] = jnp.concatenate(relrows, axis=1)
    llrow_ref[...] = jnp.concatenate(llrows, axis=1)


def _combine(hlv, hlid, llv64, llid64):
    return pl.pallas_call(
        _combine_body,
        out_shape=[
            jax.ShapeDtypeStruct((B, BEAM), jnp.float32),
            jax.ShapeDtypeStruct((B, BEAM), jnp.float32),
            jax.ShapeDtypeStruct((B, BEAM), jnp.float32),
            jax.ShapeDtypeStruct((B, BEAM), jnp.int32),
            jax.ShapeDtypeStruct((B, BEAM), jnp.int32),
        ],
    )(hlv, hlid, llv64, llid64)


def _take1(v, i_scalar):
    idx = jnp.broadcast_to(i_scalar, (16,))[:, None]
    return lax.gather(
        v, idx,
        lax.GatherDimensionNumbers(
            offset_dims=(), collapsed_slice_dims=(0,), start_index_map=(0,)),
        (1,),
        mode=lax.GatherScatterMode.PROMISE_IN_BOUNDS)


def _sc_gather(rel_eidx, ll_eidx, hl_tab, ll_tab):
    info = plsc.get_sparse_core_info()
    nc, ns, nl = info.num_cores, info.num_subcores, info.num_lanes
    nw = nc * ns
    n = rel_eidx.shape[0]
    per = n // nw
    mesh = plsc.VectorSubcoreMesh(core_axis_name="c", subcore_axis_name="s")

    @functools.partial(
        pl.kernel,
        mesh=mesh,
        out_type=[
            jax.ShapeDtypeStruct((n,), jnp.int32),
            jax.ShapeDtypeStruct((n,), jnp.int32),
            jax.ShapeDtypeStruct((n,), jnp.int32),
        ],
        scratch_types=[
            pltpu.VMEM((per,), jnp.int32),
            pltpu.VMEM((per,), jnp.int32),
            pltpu.VMEM((per,), jnp.int32),
            pltpu.VMEM((per,), jnp.int32),
            pltpu.VMEM((per,), jnp.int32),
            pltpu.VMEM((per, 128), jnp.int32),
            pltpu.VMEM((per, 128), jnp.int32),
            pltpu.VMEM((per, 128), jnp.int32),
            pltpu.VMEM((per,), jnp.int32),
            pltpu.VMEM((per,), jnp.int32),
            pltpu.VMEM((per,), jnp.int32),
            pltpu.SemaphoreType.DMA,
            pltpu.SemaphoreType.DMA,
            pltpu.SemaphoreType.DMA,
        ],
    )
    def k(rel_idx_hbm, ll_idx_hbm, hl_tab_hbm, ll_tab_hbm,
          out_rel, out_ent, out_time,
          eidx1, eidx2, row1, row2, row3, rows1, rows2, rows3,
          o1, o2, o3, sem1, sem2, sem3):
        wid = lax.axis_index("s") * nc + lax.axis_index("c")
        base = wid * per
        pltpu.sync_copy(rel_idx_hbm.at[pl.ds(base, per)], eidx1)
        pltpu.sync_copy(ll_idx_hbm.at[pl.ds(base, per)], eidx2)
        for c in range(per // nl):
            s = pl.ds(c * nl, nl)
            row1[s] = eidx1[s] >> 7
            r2 = eidx2[s] >> 7
            row2[s] = r2
            row3[s] = r2 + 1
        c1 = pltpu.async_copy(hl_tab_hbm.at[row1], rows1, sem1)
        c2 = pltpu.async_copy(ll_tab_hbm.at[row2], rows2, sem2)
        c3 = pltpu.async_copy(ll_tab_hbm.at[row3], rows3, sem3)
        c1.wait()
        c2.wait()
        c3.wait()
        iota16 = lax.broadcasted_iota(jnp.int32, (nl,), 0)
        for g in range(per // nl):
            s = pl.ds(g * nl, nl)
            e1 = eidx1[s]
            e2 = eidx2[s]
            sub1 = (e1 & 127) >> 4
            off1 = e1 & 15
            sub2 = (e2 & 127) >> 4
            off2 = e2 & 15
            a1 = jnp.zeros((nl,), jnp.int32)
            a2 = jnp.zeros((nl,), jnp.int32)
            a3 = jnp.zeros((nl,), jnp.int32)
            for kk in range(nl):
                i = g * nl + kk
                hit = iota16 == kk
                v1 = rows1[i, pl.ds(sub1[kk] * nl, nl)]
                v2 = rows2[i, pl.ds(sub2[kk] * nl, nl)]
                v3 = rows3[i, pl.ds(sub2[kk] * nl, nl)]
                a1 = jnp.where(hit, _take1(v1, off1[kk]), a1)
                a2 = jnp.where(hit, _take1(v2, off2[kk]), a2)
                a3 = jnp.where(hit, _take1(v3, off2[kk]), a3)
            o1[s] = a1
            o2[s] = a2
            o3[s] = a3
        pltpu.sync_copy(o1, out_rel.at[pl.ds(base, per)])
        pltpu.sync_copy(o2, out_ent.at[pl.ds(base, per)])
        pltpu.sync_copy(o3, out_time.at[pl.ds(base, per)])

    return k(rel_eidx, ll_eidx, hl_tab, ll_tab)


def kernel(logits_hl, hl_space, logits_ll, ll_space):
    hlv, hlid = _topk8(logits_hl, row_block=128)
    llv, llid = _topk8(logits_ll, row_block=256)

    llv64 = llv.reshape(B, HL_BEAM * LL_BEAM)
    llid64 = llid.reshape(B, HL_BEAM * LL_BEAM)

    beam, hl_g, ll_g, rel_row, ll_row = _combine(hlv, hlid, llv64, llid64)

    hl_tab = (hl_space.reshape(B, A // 128, 128, 2)
              .transpose(0, 1, 3, 2).reshape(B * A * 2 // 128, 128))
    ll_tab = (ll_space.reshape(B * HL_BEAM, A // 128, 128, 2)
              .transpose(0, 1, 3, 2).reshape(B * HL_BEAM * A * 2 // 128, 128))
    rels, ents, times = _sc_gather(
        rel_row.reshape(-1), ll_row.reshape(-1), hl_tab, ll_tab)

    return (
        beam,
        hl_g.reshape(-1),
        ll_g.reshape(-1),
        ents,
        times,
        rels,
    )
